# baseline (speedup 1.0000x reference)
import functools

import jax
import jax.numpy as jnp
from jax import lax
from jax.experimental import pallas as pl
from jax.experimental.pallas import tpu as pltpu
from jax.experimental.pallas import tpu_sc as plsc

_N, _E, _D, _H, _EXTRA, _NCLS = 10000, 320000, 128, 128, 16, 10
_LAYERS = 3

_SC_CORES = 2
_SC_SUBCORES = 16
_NW = _SC_CORES * _SC_SUBCORES
_EB = 80
_NBW = 125
_EPW = _NBW * _EB
_RING = 3
_RNDS = (_NBW + _RING - 1) // _RING
_SHIFT = 14
_MASK = (1 << _SHIFT) - 1
_ZB = 400
_NZB = _N // _ZB


_HB = _EB // 2


def _agg_body(h_hbm, pk_hbm, zero_hbm, out_hbm, pk_v, *rest):
    rows = rest[0:_RING]
    srcb = rest[_RING:2 * _RING]
    dstb = rest[2 * _RING:3 * _RING]
    acc_sh = rest[3 * _RING]
    gs = rest[3 * _RING + 1:]

    def gather_start(b):
        pltpu.async_copy(h_hbm.at[srcb[b].at[pl.ds(0, _HB)]],
                         rows[b].at[pl.ds(0, _HB)], gs[2 * b])
        pltpu.async_copy(h_hbm.at[srcb[b].at[pl.ds(_HB, _HB)]],
                         rows[b].at[pl.ds(_HB, _HB)], gs[2 * b + 1])

    def gather_wait(b):
        pltpu.make_async_copy(h_hbm.at[pl.ds(0, _HB)],
                              rows[b].at[pl.ds(0, _HB)], gs[2 * b]).wait()
        pltpu.make_async_copy(h_hbm.at[pl.ds(0, _HB)],
                              rows[b].at[pl.ds(_HB, _HB)], gs[2 * b + 1]).wait()
    cid = lax.axis_index("c")
    sid = lax.axis_index("s")
    wid = sid * _SC_CORES + cid

    pltpu.sync_copy(pk_hbm.at[wid], pk_v)

    def unpack(j, b):
        for v in range(_EB // 16):
            pkv = pk_v[pl.ds(j * _EB + v * 16, 16)]
            srcb[b][pl.ds(v * 16, 16)] = lax.shift_right_logical(pkv, _SHIFT)
            dstb[b][pl.ds(v * 16, 16)] = lax.bitwise_and(pkv, _MASK)

    def zstep(k, carry):
        blk = sid + _SC_SUBCORES * k

        @pl.when(blk < _NZB)
        def _():
            off = pl.multiple_of(blk * _ZB, 8)
            pltpu.sync_copy(zero_hbm.at[pl.ds(off, _ZB)],
                            acc_sh.at[pl.ds(off, _ZB)])
        return carry

    lax.fori_loop(0, (_NZB + _SC_SUBCORES - 1) // _SC_SUBCORES, zstep, 0)
    plsc.subcore_barrier()

    for b in range(_RING):
        unpack(b, b)
        gather_start(b)

    def outer(g, carry):
        for b in range(_RING):
            j = g * _RING + b

            @pl.when(j < _NBW)
            def _():
                gather_wait(b)
                pltpu.sync_copy(rows[b], acc_sh.at[dstb[b]], add=True)
                nj = j + _RING

                @pl.when(nj < _NBW)
                def _():
                    unpack(nj, b)
                    gather_start(b)
        return carry

    lax.fori_loop(0, _RNDS, outer, 0)
    plsc.subcore_barrier()

    def ostep(k, carry):
        blk = sid + _SC_SUBCORES * k

        @pl.when(blk < _NZB)
        def _():
            off = pl.multiple_of(blk * _ZB, 8)
            pltpu.sync_copy(acc_sh.at[pl.ds(off, _ZB)],
                            out_hbm.at[cid, pl.ds(off, _ZB)])
        return carry

    lax.fori_loop(0, (_NZB + _SC_SUBCORES - 1) // _SC_SUBCORES, ostep, 0)


_agg = pl.kernel(
    _agg_body,
    out_type=jax.ShapeDtypeStruct((_SC_CORES, _N, _D), jnp.float32),
    mesh=plsc.VectorSubcoreMesh(core_axis_name="c", subcore_axis_name="s"),
    scratch_types=[
        pltpu.VMEM((_EPW,), jnp.int32),
    ] + [pltpu.VMEM((_EB, _D), jnp.float32) for _ in range(_RING)]
    + [pltpu.VMEM((_EB,), jnp.int32) for _ in range(2 * _RING)] + [
        pltpu.VMEM_SHARED((_N, _D), jnp.float32),
    ] + [pltpu.SemaphoreType.DMA for _ in range(2 * _RING)],
)


def _ln(z, g, b):
    m = jnp.mean(z, axis=-1, keepdims=True)
    v = jnp.mean((z - m) * (z - m), axis=-1, keepdims=True)
    return (z - m) * lax.rsqrt(v + 1e-5) * g + b


_BR = 1000


def _mlp_body(h_ref, a0_ref, a1_ref, eps_ref, w1_ref, b1_ref, g1_ref,
              be1_ref, w2_ref, b2_ref, gn_ref, bn_ref, out_ref):
    rst = eps_ref[0, 0] * h_ref[...] + a0_ref[0] + a1_ref[0]
    z = jnp.dot(rst, w1_ref[...], preferred_element_type=jnp.float32) + b1_ref[...]
    z = jnp.maximum(_ln(z, g1_ref[...], be1_ref[...]), 0.0)
    z = jnp.dot(z, w2_ref[...], preferred_element_type=jnp.float32) + b2_ref[...]
    out_ref[...] = jnp.maximum(_ln(z, gn_ref[...], bn_ref[...]), 0.0)


def _bcast(shape):
    return pl.BlockSpec(shape, lambda i: (0, 0))


_mlp = pl.pallas_call(
    _mlp_body,
    grid=(_N // _BR,),
    in_specs=[
        pl.BlockSpec((_BR, _D), lambda i: (i, 0)),
        pl.BlockSpec((1, _BR, _D), lambda i: (0, i, 0)),
        pl.BlockSpec((1, _BR, _D), lambda i: (1, i, 0)),
        _bcast((1, 1)),
        _bcast((_D, _H)),
        _bcast((1, _H)),
        _bcast((1, _H)),
        _bcast((1, _H)),
        _bcast((_H, _H)),
        _bcast((1, _H)),
        _bcast((1, _H)),
        _bcast((1, _H)),
    ],
    out_specs=pl.BlockSpec((_BR, _H), lambda i: (i, 0)),
    out_shape=jax.ShapeDtypeStruct((_N, _H), jnp.float32),
)


def _head_body(h_ref, desc_ref, wh_ref, wd_ref, b1_ref, g_ref, be_ref,
               w2_ref, b2_ref, out_ref):
    hg = jnp.sum(h_ref[...], axis=0, keepdims=True)
    y = (jnp.dot(hg, wh_ref[...], preferred_element_type=jnp.float32)
         + jnp.dot(desc_ref[...], wd_ref[...], preferred_element_type=jnp.float32)
         + b1_ref[...])
    y = jnp.maximum(_ln(y, g_ref[...], be_ref[...]), 0.0)
    out_ref[...] = jnp.dot(y, w2_ref[...], preferred_element_type=jnp.float32) + b2_ref[...]


_head = pl.pallas_call(
    _head_body,
    out_shape=jax.ShapeDtypeStruct((1, _NCLS), jnp.float32),
)


def kernel(x, edge_index, desc, params):
    pk = jnp.bitwise_or(
        jnp.left_shift(edge_index[0], _SHIFT),
        edge_index[1]).reshape(_NW, _EPW)
    zeros_nd = jnp.zeros((_N, _D), jnp.float32)
    h = x
    for l in range(_LAYERS):
        agg = _agg(h, pk, zeros_nd)
        eps1 = (1.0 + params['eps_%d' % l]).reshape(1, 1)
        h = _mlp(h, agg, agg, eps1,
                 params['W1_%d' % l], params['b1_%d' % l].reshape(1, _H),
                 params['g1_%d' % l].reshape(1, _H),
                 params['be1_%d' % l].reshape(1, _H),
                 params['W2_%d' % l], params['b2_%d' % l].reshape(1, _H),
                 params['gn_%d' % l].reshape(1, _H),
                 params['bn_%d' % l].reshape(1, _H))
    out = _head(h, desc,
                params['fc1_W'][:_H], params['fc1_W'][_H:],
                params['fc1_b'].reshape(1, _H),
                params['n1_g'].reshape(1, _H), params['n1_b'].reshape(1, _H),
                params['fc2_W'], params['fc2_b'].reshape(1, _NCLS))
    return out


# speedup vs baseline: 1.0108x; 1.0108x over previous
import functools

import jax
import jax.numpy as jnp
from jax import lax
from jax.experimental import pallas as pl
from jax.experimental.pallas import tpu as pltpu
from jax.experimental.pallas import tpu_sc as plsc

_N, _E, _D, _H, _EXTRA, _NCLS = 10000, 320000, 128, 128, 16, 10
_LAYERS = 3

_SC_CORES = 2
_SC_SUBCORES = 16
_NW = _SC_CORES * _SC_SUBCORES
_EB = 80
_NBW = 125
_EPW = _NBW * _EB
_RING = 3
_RNDS = (_NBW + _RING - 1) // _RING
_SHIFT = 14
_MASK = (1 << _SHIFT) - 1
_ZB = 400
_NZB = _N // _ZB


_HB = _EB // 2


def _agg_body(h_hbm, pk_hbm, zero_hbm, out_hbm, pk_v, *rest):
    rows = rest[0:_RING]
    srcb = rest[_RING:2 * _RING]
    dstb = rest[2 * _RING:3 * _RING]
    acc_sh = rest[3 * _RING]
    gs = rest[3 * _RING + 1:]

    def gather_start(b):
        pltpu.async_copy(h_hbm.at[srcb[b]], rows[b], gs[b])

    def gather_wait(b):
        pltpu.make_async_copy(h_hbm.at[pl.ds(0, _EB)], rows[b], gs[b]).wait()
    cid = lax.axis_index("c")
    sid = lax.axis_index("s")
    wid = sid * _SC_CORES + cid

    pltpu.async_copy(pk_hbm.at[wid], pk_v, gs[0])

    def unpack(j, b):
        for v in range(_EB // 16):
            pkv = pk_v[pl.ds(j * _EB + v * 16, 16)]
            srcb[b][pl.ds(v * 16, 16)] = lax.shift_right_logical(pkv, _SHIFT)
            dstb[b][pl.ds(v * 16, 16)] = lax.bitwise_and(pkv, _MASK)

    def zstep(k, carry):
        blk = sid + _SC_SUBCORES * k

        @pl.when(blk < _NZB)
        def _():
            off = pl.multiple_of(blk * _ZB, 8)
            pltpu.sync_copy(zero_hbm.at[pl.ds(off, _ZB)],
                            acc_sh.at[pl.ds(off, _ZB)])
        return carry

    lax.fori_loop(0, (_NZB + _SC_SUBCORES - 1) // _SC_SUBCORES, zstep, 0)
    pltpu.make_async_copy(pk_hbm.at[0], pk_v, gs[0]).wait()
    plsc.subcore_barrier()

    for b in range(_RING):
        unpack(b, b)
        gather_start(b)

    def outer(g, carry):
        for b in range(_RING):
            j = g * _RING + b

            @pl.when(j < _NBW)
            def _():
                gather_wait(b)
                pltpu.sync_copy(rows[b], acc_sh.at[dstb[b]], add=True)
                nj = j + _RING

                @pl.when(nj < _NBW)
                def _():
                    unpack(nj, b)
                    gather_start(b)
        return carry

    lax.fori_loop(0, _RNDS, outer, 0)
    plsc.subcore_barrier()

    def ostep(k, carry):
        blk = sid + _SC_SUBCORES * k

        @pl.when(blk < _NZB)
        def _():
            off = pl.multiple_of(blk * _ZB, 8)
            pltpu.sync_copy(acc_sh.at[pl.ds(off, _ZB)],
                            out_hbm.at[cid, pl.ds(off, _ZB)])
        return carry

    lax.fori_loop(0, (_NZB + _SC_SUBCORES - 1) // _SC_SUBCORES, ostep, 0)


_agg = pl.kernel(
    _agg_body,
    out_type=jax.ShapeDtypeStruct((_SC_CORES, _N, _D), jnp.float32),
    mesh=plsc.VectorSubcoreMesh(core_axis_name="c", subcore_axis_name="s"),
    scratch_types=[
        pltpu.VMEM((_EPW,), jnp.int32),
    ] + [pltpu.VMEM((_EB, _D), jnp.float32) for _ in range(_RING)]
    + [pltpu.VMEM((_EB,), jnp.int32) for _ in range(2 * _RING)] + [
        pltpu.VMEM_SHARED((_N, _D), jnp.float32),
    ] + [pltpu.SemaphoreType.DMA for _ in range(2 * _RING)],
)


def _ln(z, g, b):
    m = jnp.mean(z, axis=-1, keepdims=True)
    v = jnp.mean((z - m) * (z - m), axis=-1, keepdims=True)
    return (z - m) * lax.rsqrt(v + 1e-5) * g + b


_BR = 1000


def _mlp_body(h_ref, a0_ref, a1_ref, eps_ref, w1_ref, b1_ref, g1_ref,
              be1_ref, w2_ref, b2_ref, gn_ref, bn_ref, out_ref):
    rst = eps_ref[0, 0] * h_ref[...] + a0_ref[0] + a1_ref[0]
    z = jnp.dot(rst, w1_ref[...], preferred_element_type=jnp.float32) + b1_ref[...]
    z = jnp.maximum(_ln(z, g1_ref[...], be1_ref[...]), 0.0)
    z = jnp.dot(z, w2_ref[...], preferred_element_type=jnp.float32) + b2_ref[...]
    out_ref[...] = jnp.maximum(_ln(z, gn_ref[...], bn_ref[...]), 0.0)


def _bcast(shape):
    return pl.BlockSpec(shape, lambda i: (0, 0))


_mlp = pl.pallas_call(
    _mlp_body,
    grid=(_N // _BR,),
    in_specs=[
        pl.BlockSpec((_BR, _D), lambda i: (i, 0)),
        pl.BlockSpec((1, _BR, _D), lambda i: (0, i, 0)),
        pl.BlockSpec((1, _BR, _D), lambda i: (1, i, 0)),
        _bcast((1, 1)),
        _bcast((_D, _H)),
        _bcast((1, _H)),
        _bcast((1, _H)),
        _bcast((1, _H)),
        _bcast((_H, _H)),
        _bcast((1, _H)),
        _bcast((1, _H)),
        _bcast((1, _H)),
    ],
    out_specs=pl.BlockSpec((_BR, _H), lambda i: (i, 0)),
    out_shape=jax.ShapeDtypeStruct((_N, _H), jnp.float32),
)


def _head_body(h_ref, desc_ref, wh_ref, wd_ref, b1_ref, g_ref, be_ref,
               w2_ref, b2_ref, out_ref):
    hg = jnp.sum(h_ref[...], axis=0, keepdims=True)
    y = (jnp.dot(hg, wh_ref[...], preferred_element_type=jnp.float32)
         + jnp.dot(desc_ref[...], wd_ref[...], preferred_element_type=jnp.float32)
         + b1_ref[...])
    y = jnp.maximum(_ln(y, g_ref[...], be_ref[...]), 0.0)
    out_ref[...] = jnp.dot(y, w2_ref[...], preferred_element_type=jnp.float32) + b2_ref[...]


_head = pl.pallas_call(
    _head_body,
    out_shape=jax.ShapeDtypeStruct((1, _NCLS), jnp.float32),
)


def kernel(x, edge_index, desc, params):
    pk = jnp.bitwise_or(
        jnp.left_shift(edge_index[0], _SHIFT),
        edge_index[1]).reshape(_NW, _EPW)
    zeros_nd = jnp.zeros((_N, _D), jnp.float32)
    h = x
    for l in range(_LAYERS):
        agg = _agg(h, pk, zeros_nd)
        eps1 = (1.0 + params['eps_%d' % l]).reshape(1, 1)
        h = _mlp(h, agg, agg, eps1,
                 params['W1_%d' % l], params['b1_%d' % l].reshape(1, _H),
                 params['g1_%d' % l].reshape(1, _H),
                 params['be1_%d' % l].reshape(1, _H),
                 params['W2_%d' % l], params['b2_%d' % l].reshape(1, _H),
                 params['gn_%d' % l].reshape(1, _H),
                 params['bn_%d' % l].reshape(1, _H))
    out = _head(h, desc,
                params['fc1_W'][:_H], params['fc1_W'][_H:],
                params['fc1_b'].reshape(1, _H),
                params['n1_g'].reshape(1, _H), params['n1_b'].reshape(1, _H),
                params['fc2_W'], params['fc2_b'].reshape(1, _NCLS))
    return out


# speedup vs baseline: 1.0387x; 1.0276x over previous
import functools

import jax
import jax.numpy as jnp
from jax import lax
from jax.experimental import pallas as pl
from jax.experimental.pallas import tpu as pltpu
from jax.experimental.pallas import tpu_sc as plsc

_N, _E, _D, _H, _EXTRA, _NCLS = 10000, 320000, 128, 128, 16, 10
_LAYERS = 3

_SC_CORES = 2
_SC_SUBCORES = 16
_NW = _SC_CORES * _SC_SUBCORES
_EB = 80
_NBW = 125
_EPW = _NBW * _EB
_RING = 3
_RNDS = (_NBW + _RING - 1) // _RING
_SHIFT = 14
_MASK = (1 << _SHIFT) - 1
_ZB = 400
_NZB = _N // _ZB


_HB = _EB // 2


def _agg_body(h_hbm, pk_hbm, zero_hbm, out_hbm, pk_v, *rest):
    rows = rest[0:_RING]
    srcb = rest[_RING:2 * _RING]
    dstb = rest[2 * _RING:3 * _RING]
    acc_sh = rest[3 * _RING]
    gs = rest[3 * _RING + 1:]

    def gather_start(b):
        pltpu.async_copy(h_hbm.at[srcb[b]], rows[b], gs[b])

    def gather_wait(b):
        pltpu.make_async_copy(h_hbm.at[pl.ds(0, _EB)], rows[b], gs[b]).wait()
    cid = lax.axis_index("c")
    sid = lax.axis_index("s")
    wid = sid * _SC_CORES + cid

    pltpu.async_copy(pk_hbm.at[wid], pk_v, gs[0])

    def unpack(j, b):
        for v in range(_EB // 16):
            pkv = pk_v[pl.ds(j * _EB + v * 16, 16)]
            srcb[b][pl.ds(v * 16, 16)] = lax.shift_right_logical(pkv, _SHIFT)
            dstb[b][pl.ds(v * 16, 16)] = lax.bitwise_and(pkv, _MASK)

    def zstep(k, carry):
        blk = sid + _SC_SUBCORES * k

        @pl.when(blk < _NZB)
        def _():
            off = pl.multiple_of(blk * _ZB, 8)
            pltpu.sync_copy(zero_hbm.at[pl.ds(off, _ZB)],
                            acc_sh.at[pl.ds(off, _ZB)])
        return carry

    lax.fori_loop(0, (_NZB + _SC_SUBCORES - 1) // _SC_SUBCORES, zstep, 0)
    pltpu.make_async_copy(pk_hbm.at[0], pk_v, gs[0]).wait()
    plsc.subcore_barrier()

    for b in range(_RING):
        unpack(b, b)
        gather_start(b)

    def outer(g, carry):
        for b in range(_RING):
            j = g * _RING + b

            @pl.when(j < _NBW)
            def _():
                gather_wait(b)
                pltpu.sync_copy(rows[b], acc_sh.at[dstb[b]], add=True)
                nj = j + _RING

                @pl.when(nj < _NBW)
                def _():
                    unpack(nj, b)
                    gather_start(b)
        return carry

    lax.fori_loop(0, _RNDS, outer, 0)
    plsc.subcore_barrier()

    def ostep(k, carry):
        blk = sid + _SC_SUBCORES * k

        @pl.when(blk < _NZB)
        def _():
            off = pl.multiple_of(blk * _ZB, 8)
            pltpu.sync_copy(acc_sh.at[pl.ds(off, _ZB)],
                            out_hbm.at[cid, pl.ds(off, _ZB)])
        return carry

    lax.fori_loop(0, (_NZB + _SC_SUBCORES - 1) // _SC_SUBCORES, ostep, 0)


_agg = pl.kernel(
    _agg_body,
    out_type=jax.ShapeDtypeStruct((_SC_CORES, _N, _D), jnp.float32),
    mesh=plsc.VectorSubcoreMesh(core_axis_name="c", subcore_axis_name="s"),
    scratch_types=[
        pltpu.VMEM((_EPW,), jnp.int32),
    ] + [pltpu.VMEM((_EB, _D), jnp.float32) for _ in range(_RING)]
    + [pltpu.VMEM((_EB,), jnp.int32) for _ in range(2 * _RING)] + [
        pltpu.VMEM_SHARED((_N, _D), jnp.float32),
    ] + [pltpu.SemaphoreType.DMA for _ in range(2 * _RING)],
)


def _ln(z, g, b):
    m = jnp.mean(z, axis=-1, keepdims=True)
    v = jnp.mean((z - m) * (z - m), axis=-1, keepdims=True)
    return (z - m) * lax.rsqrt(v + 1e-5) * g + b


_BR = 2000
_NBLK = _N // _BR


def _mlp_body(h_ref, a0_ref, a1_ref, eps_ref, w1_ref, b1_ref, g1_ref,
              be1_ref, w2_ref, b2_ref, gn_ref, bn_ref, out_ref, psum_ref):
    rst = eps_ref[0, 0] * h_ref[...] + a0_ref[0] + a1_ref[0]
    z = jnp.dot(rst, w1_ref[...], preferred_element_type=jnp.float32) + b1_ref[...]
    z = jnp.maximum(_ln(z, g1_ref[...], be1_ref[...]), 0.0)
    z = jnp.dot(z, w2_ref[...], preferred_element_type=jnp.float32) + b2_ref[...]
    h_out = jnp.maximum(_ln(z, gn_ref[...], bn_ref[...]), 0.0)
    out_ref[...] = h_out
    psum_ref[0] = jnp.sum(h_out, axis=0, keepdims=True)


def _bcast(shape):
    return pl.BlockSpec(shape, lambda i: (0, 0))


_mlp = pl.pallas_call(
    _mlp_body,
    grid=(_NBLK,),
    in_specs=[
        pl.BlockSpec((_BR, _D), lambda i: (i, 0)),
        pl.BlockSpec((1, _BR, _D), lambda i: (0, i, 0)),
        pl.BlockSpec((1, _BR, _D), lambda i: (1, i, 0)),
        _bcast((1, 1)),
        _bcast((_D, _H)),
        _bcast((1, _H)),
        _bcast((1, _H)),
        _bcast((1, _H)),
        _bcast((_H, _H)),
        _bcast((1, _H)),
        _bcast((1, _H)),
        _bcast((1, _H)),
    ],
    out_specs=[pl.BlockSpec((_BR, _H), lambda i: (i, 0)),
               pl.BlockSpec((1, 1, _H), lambda i: (i, 0, 0))],
    out_shape=[jax.ShapeDtypeStruct((_N, _H), jnp.float32),
               jax.ShapeDtypeStruct((_NBLK, 1, _H), jnp.float32)],
)


def _head_body(h_ref, desc_ref, wh_ref, wd_ref, b1_ref, g_ref, be_ref,
               w2_ref, b2_ref, out_ref):
    hg = jnp.sum(h_ref[...], axis=0, keepdims=True)
    y = (jnp.dot(hg, wh_ref[...], preferred_element_type=jnp.float32)
         + jnp.dot(desc_ref[...], wd_ref[...], preferred_element_type=jnp.float32)
         + b1_ref[...])
    y = jnp.maximum(_ln(y, g_ref[...], be_ref[...]), 0.0)
    out_ref[...] = jnp.dot(y, w2_ref[...], preferred_element_type=jnp.float32) + b2_ref[...]


_head = pl.pallas_call(
    _head_body,
    out_shape=jax.ShapeDtypeStruct((1, _NCLS), jnp.float32),
)


def kernel(x, edge_index, desc, params):
    pk = jnp.bitwise_or(
        jnp.left_shift(edge_index[0], _SHIFT),
        edge_index[1]).reshape(_NW, _EPW)
    zeros_nd = jnp.zeros((_N, _D), jnp.float32)
    h = x
    psum = None
    for l in range(_LAYERS):
        agg = _agg(h, pk, zeros_nd)
        eps1 = (1.0 + params['eps_%d' % l]).reshape(1, 1)
        h, psum = _mlp(h, agg, agg, eps1,
                 params['W1_%d' % l], params['b1_%d' % l].reshape(1, _H),
                 params['g1_%d' % l].reshape(1, _H),
                 params['be1_%d' % l].reshape(1, _H),
                 params['W2_%d' % l], params['b2_%d' % l].reshape(1, _H),
                 params['gn_%d' % l].reshape(1, _H),
                 params['bn_%d' % l].reshape(1, _H))
    out = _head(psum.reshape(_NBLK, _H), desc,
                params['fc1_W'][:_H], params['fc1_W'][_H:],
                params['fc1_b'].reshape(1, _H),
                params['n1_g'].reshape(1, _H), params['n1_b'].reshape(1, _H),
                params['fc2_W'], params['fc2_b'].reshape(1, _NCLS))
    return out


# speedup vs baseline: 1.0488x; 1.0098x over previous
import functools

import jax
import jax.numpy as jnp
from jax import lax
from jax.experimental import pallas as pl
from jax.experimental.pallas import tpu as pltpu
from jax.experimental.pallas import tpu_sc as plsc

_N, _E, _D, _H, _EXTRA, _NCLS = 10000, 320000, 128, 128, 16, 10
_LAYERS = 3

_SC_CORES = 2
_SC_SUBCORES = 16
_NW = _SC_CORES * _SC_SUBCORES
_EB = 80
_NBW = 125
_EPW = _NBW * _EB
_RING = 3
_RNDS = (_NBW + _RING - 1) // _RING
_SHIFT = 14
_MASK = (1 << _SHIFT) - 1
_ZB = 400
_NZB = _N // _ZB


_HB = _EB // 2


_ZR = 40


def _agg_body(h_hbm, pk_hbm, out_hbm, pk_v, *rest):
    rows = rest[0:_RING]
    srcb = rest[_RING:2 * _RING]
    dstb = rest[2 * _RING:3 * _RING]
    zbuf = rest[3 * _RING]
    acc_sh = rest[3 * _RING + 1]
    gs = rest[3 * _RING + 2:]

    def gather_start(b):
        pltpu.async_copy(h_hbm.at[srcb[b]], rows[b], gs[b])

    def gather_wait(b):
        pltpu.make_async_copy(h_hbm.at[pl.ds(0, _EB)], rows[b], gs[b]).wait()
    cid = lax.axis_index("c")
    sid = lax.axis_index("s")
    wid = sid * _SC_CORES + cid

    pltpu.async_copy(pk_hbm.at[wid], pk_v, gs[0])

    def unpack(j, b):
        for v in range(_EB // 16):
            pkv = pk_v[pl.ds(j * _EB + v * 16, 16)]
            srcb[b][pl.ds(v * 16, 16)] = lax.shift_right_logical(pkv, _SHIFT)
            dstb[b][pl.ds(v * 16, 16)] = lax.bitwise_and(pkv, _MASK)

    zv = jnp.zeros((16,), jnp.float32)

    def zfill(i, carry):
        r = i // (_D // 16)
        c = lax.rem(i, _D // 16)
        zbuf[r, pl.ds(c * 16, 16)] = zv
        return carry

    lax.fori_loop(0, _ZR * _D // 16, zfill, 0)

    def zstep(k, carry):
        blk = sid + _SC_SUBCORES * k

        @pl.when(blk < _NZB)
        def _():
            for c in range(_ZB // _ZR):
                off = pl.multiple_of(blk * _ZB + c * _ZR, 8)
                pltpu.sync_copy(zbuf, acc_sh.at[pl.ds(off, _ZR)])
        return carry

    lax.fori_loop(0, (_NZB + _SC_SUBCORES - 1) // _SC_SUBCORES, zstep, 0)
    pltpu.make_async_copy(pk_hbm.at[0], pk_v, gs[0]).wait()
    plsc.subcore_barrier()

    for b in range(_RING):
        unpack(b, b)
        gather_start(b)

    def outer(g, carry):
        for b in range(_RING):
            j = g * _RING + b

            @pl.when(j < _NBW)
            def _():
                gather_wait(b)
                pltpu.sync_copy(rows[b], acc_sh.at[dstb[b]], add=True)
                nj = j + _RING

                @pl.when(nj < _NBW)
                def _():
                    unpack(nj, b)
                    gather_start(b)
        return carry

    lax.fori_loop(0, _RNDS, outer, 0)
    plsc.subcore_barrier()

    def ostep(k, carry):
        blk = sid + _SC_SUBCORES * k

        @pl.when(blk < _NZB)
        def _():
            off = pl.multiple_of(blk * _ZB, 8)
            pltpu.sync_copy(acc_sh.at[pl.ds(off, _ZB)],
                            out_hbm.at[cid, pl.ds(off, _ZB)])
        return carry

    lax.fori_loop(0, (_NZB + _SC_SUBCORES - 1) // _SC_SUBCORES, ostep, 0)


_agg = pl.kernel(
    _agg_body,
    out_type=jax.ShapeDtypeStruct((_SC_CORES, _N, _D), jnp.float32),
    mesh=plsc.VectorSubcoreMesh(core_axis_name="c", subcore_axis_name="s"),
    scratch_types=[
        pltpu.VMEM((_EPW,), jnp.int32),
    ] + [pltpu.VMEM((_EB, _D), jnp.float32) for _ in range(_RING)]
    + [pltpu.VMEM((_EB,), jnp.int32) for _ in range(2 * _RING)] + [
        pltpu.VMEM((_ZR, _D), jnp.float32),
        pltpu.VMEM_SHARED((_N, _D), jnp.float32),
    ] + [pltpu.SemaphoreType.DMA for _ in range(2 * _RING)],
)


def _ln(z, g, b):
    m = jnp.mean(z, axis=-1, keepdims=True)
    v = jnp.mean((z - m) * (z - m), axis=-1, keepdims=True)
    return (z - m) * lax.rsqrt(v + 1e-5) * g + b


_BR = 2000
_NBLK = _N // _BR


def _mlp_body(h_ref, a0_ref, a1_ref, eps_ref, w1_ref, b1_ref, g1_ref,
              be1_ref, w2_ref, b2_ref, gn_ref, bn_ref, out_ref, psum_ref):
    rst = eps_ref[0, 0] * h_ref[...] + a0_ref[0] + a1_ref[0]
    z = jnp.dot(rst, w1_ref[...], preferred_element_type=jnp.float32) + b1_ref[...]
    z = jnp.maximum(_ln(z, g1_ref[...], be1_ref[...]), 0.0)
    z = jnp.dot(z, w2_ref[...], preferred_element_type=jnp.float32) + b2_ref[...]
    h_out = jnp.maximum(_ln(z, gn_ref[...], bn_ref[...]), 0.0)
    out_ref[...] = h_out
    psum_ref[0] = jnp.sum(h_out, axis=0, keepdims=True)


def _bcast(shape):
    return pl.BlockSpec(shape, lambda i: (0, 0))


_mlp = pl.pallas_call(
    _mlp_body,
    grid=(_NBLK,),
    in_specs=[
        pl.BlockSpec((_BR, _D), lambda i: (i, 0)),
        pl.BlockSpec((1, _BR, _D), lambda i: (0, i, 0)),
        pl.BlockSpec((1, _BR, _D), lambda i: (1, i, 0)),
        _bcast((1, 1)),
        _bcast((_D, _H)),
        _bcast((1, _H)),
        _bcast((1, _H)),
        _bcast((1, _H)),
        _bcast((_H, _H)),
        _bcast((1, _H)),
        _bcast((1, _H)),
        _bcast((1, _H)),
    ],
    out_specs=[pl.BlockSpec((_BR, _H), lambda i: (i, 0)),
               pl.BlockSpec((1, 1, _H), lambda i: (i, 0, 0))],
    out_shape=[jax.ShapeDtypeStruct((_N, _H), jnp.float32),
               jax.ShapeDtypeStruct((_NBLK, 1, _H), jnp.float32)],
)


def _head_body(h_ref, desc_ref, wh_ref, wd_ref, b1_ref, g_ref, be_ref,
               w2_ref, b2_ref, out_ref):
    hg = jnp.sum(h_ref[...], axis=0, keepdims=True)
    y = (jnp.dot(hg, wh_ref[...], preferred_element_type=jnp.float32)
         + jnp.dot(desc_ref[...], wd_ref[...], preferred_element_type=jnp.float32)
         + b1_ref[...])
    y = jnp.maximum(_ln(y, g_ref[...], be_ref[...]), 0.0)
    out_ref[...] = jnp.dot(y, w2_ref[...], preferred_element_type=jnp.float32) + b2_ref[...]


_head = pl.pallas_call(
    _head_body,
    out_shape=jax.ShapeDtypeStruct((1, _NCLS), jnp.float32),
)


def kernel(x, edge_index, desc, params):
    pk = jnp.bitwise_or(
        jnp.left_shift(edge_index[0], _SHIFT),
        edge_index[1]).reshape(_NW, _EPW)
    h = x
    psum = None
    for l in range(_LAYERS):
        agg = _agg(h, pk)
        eps1 = (1.0 + params['eps_%d' % l]).reshape(1, 1)
        h, psum = _mlp(h, agg, agg, eps1,
                 params['W1_%d' % l], params['b1_%d' % l].reshape(1, _H),
                 params['g1_%d' % l].reshape(1, _H),
                 params['be1_%d' % l].reshape(1, _H),
                 params['W2_%d' % l], params['b2_%d' % l].reshape(1, _H),
                 params['gn_%d' % l].reshape(1, _H),
                 params['bn_%d' % l].reshape(1, _H))
    out = _head(psum.reshape(_NBLK, _H), desc,
                params['fc1_W'][:_H], params['fc1_W'][_H:],
                params['fc1_b'].reshape(1, _H),
                params['n1_g'].reshape(1, _H), params['n1_b'].reshape(1, _H),
                params['fc2_W'], params['fc2_b'].reshape(1, _NCLS))
    return out


# speedup vs baseline: 1.0495x; 1.0006x over previous
import jax
import jax.numpy as jnp
from jax import lax
from jax.experimental import pallas as pl
from jax.experimental.pallas import tpu as pltpu
from jax.experimental.pallas import tpu_sc as plsc

_N, _E, _D, _H, _EXTRA, _NCLS = 10000, 320000, 128, 128, 16, 10
_LAYERS = 3

_SC_CORES = 2
_SC_SUBCORES = 16
_NW = _SC_CORES * _SC_SUBCORES
_EB = 80
_NBW = 125
_EPW = _NBW * _EB
_RING = 3
_RNDS = (_NBW + _RING - 1) // _RING
_SHIFT = 14
_MASK = (1 << _SHIFT) - 1
_ZB = 400
_NZB = _N // _ZB


_ZR = 40


def _agg_body(h_hbm, pk_hbm, out_hbm, pk_v, *rest):
    rows = rest[0:_RING]
    srcb = rest[_RING:2 * _RING]
    dstb = rest[2 * _RING:3 * _RING]
    zbuf = rest[3 * _RING]
    acc_sh = rest[3 * _RING + 1]
    gs = rest[3 * _RING + 2:]

    def gather_start(b):
        pltpu.async_copy(h_hbm.at[srcb[b]], rows[b], gs[b])

    def gather_wait(b):
        pltpu.make_async_copy(h_hbm.at[pl.ds(0, _EB)], rows[b], gs[b]).wait()
    cid = lax.axis_index("c")
    sid = lax.axis_index("s")
    wid = sid * _SC_CORES + cid

    pltpu.async_copy(pk_hbm.at[wid], pk_v, gs[0])

    def unpack(j, b):
        for v in range(_EB // 16):
            pkv = pk_v[pl.ds(j * _EB + v * 16, 16)]
            srcb[b][pl.ds(v * 16, 16)] = lax.shift_right_logical(pkv, _SHIFT)
            dstb[b][pl.ds(v * 16, 16)] = lax.bitwise_and(pkv, _MASK)

    zv = jnp.zeros((16,), jnp.float32)

    def zfill(i, carry):
        r = i // (_D // 16)
        c = lax.rem(i, _D // 16)
        zbuf[r, pl.ds(c * 16, 16)] = zv
        return carry

    lax.fori_loop(0, _ZR * _D // 16, zfill, 0)

    def zstep(k, carry):
        blk = sid + _SC_SUBCORES * k

        @pl.when(blk < _NZB)
        def _():
            for c in range(_ZB // _ZR):
                off = pl.multiple_of(blk * _ZB + c * _ZR, 8)
                pltpu.sync_copy(zbuf, acc_sh.at[pl.ds(off, _ZR)])
        return carry

    lax.fori_loop(0, (_NZB + _SC_SUBCORES - 1) // _SC_SUBCORES, zstep, 0)
    pltpu.make_async_copy(pk_hbm.at[0], pk_v, gs[0]).wait()
    plsc.subcore_barrier()

    for b in range(_RING):
        unpack(b, b)
        gather_start(b)

    def outer(g, carry):
        for b in range(_RING):
            j = g * _RING + b

            @pl.when(j < _NBW)
            def _():
                gather_wait(b)
                pltpu.sync_copy(rows[b], acc_sh.at[dstb[b]], add=True)
                nj = j + _RING

                @pl.when(nj < _NBW)
                def _():
                    unpack(nj, b)
                    gather_start(b)
        return carry

    lax.fori_loop(0, _RNDS, outer, 0)
    plsc.subcore_barrier()

    def ostep(k, carry):
        blk = sid + _SC_SUBCORES * k

        @pl.when(blk < _NZB)
        def _():
            off = pl.multiple_of(blk * _ZB, 8)
            pltpu.sync_copy(acc_sh.at[pl.ds(off, _ZB)],
                            out_hbm.at[cid, pl.ds(off, _ZB)])
        return carry

    lax.fori_loop(0, (_NZB + _SC_SUBCORES - 1) // _SC_SUBCORES, ostep, 0)


_agg = pl.kernel(
    _agg_body,
    out_type=jax.ShapeDtypeStruct((_SC_CORES, _N, _D), jnp.float32),
    mesh=plsc.VectorSubcoreMesh(core_axis_name="c", subcore_axis_name="s"),
    scratch_types=[
        pltpu.VMEM((_EPW,), jnp.int32),
    ] + [pltpu.VMEM((_EB, _D), jnp.float32) for _ in range(_RING)]
    + [pltpu.VMEM((_EB,), jnp.int32) for _ in range(2 * _RING)] + [
        pltpu.VMEM((_ZR, _D), jnp.float32),
        pltpu.VMEM_SHARED((_N, _D), jnp.float32),
    ] + [pltpu.SemaphoreType.DMA for _ in range(2 * _RING)],
)


def _ln(z, g, b):
    m = jnp.mean(z, axis=-1, keepdims=True)
    v = jnp.mean((z - m) * (z - m), axis=-1, keepdims=True)
    return (z - m) * lax.rsqrt(v + 1e-5) * g + b


_BR = 2000
_NBLK = _N // _BR


def _mlp_body(h_ref, a0_ref, a1_ref, eps_ref, w1_ref, b1_ref, g1_ref,
              be1_ref, w2_ref, b2_ref, gn_ref, bn_ref, out_ref, psum_ref):
    rst = eps_ref[0, 0] * h_ref[...] + a0_ref[0] + a1_ref[0]
    z = jnp.dot(rst, w1_ref[...], preferred_element_type=jnp.float32) + b1_ref[...]
    z = jnp.maximum(_ln(z, g1_ref[...], be1_ref[...]), 0.0)
    z = jnp.dot(z, w2_ref[...], preferred_element_type=jnp.float32) + b2_ref[...]
    h_out = jnp.maximum(_ln(z, gn_ref[...], bn_ref[...]), 0.0)
    out_ref[...] = h_out
    psum_ref[0] = jnp.sum(h_out, axis=0, keepdims=True)


def _bcast(shape):
    return pl.BlockSpec(shape, lambda i: (0, 0))


_mlp = pl.pallas_call(
    _mlp_body,
    grid=(_NBLK,),
    in_specs=[
        pl.BlockSpec((_BR, _D), lambda i: (i, 0)),
        pl.BlockSpec((1, _BR, _D), lambda i: (0, i, 0)),
        pl.BlockSpec((1, _BR, _D), lambda i: (1, i, 0)),
        _bcast((1, 1)),
        _bcast((_D, _H)),
        _bcast((1, _H)),
        _bcast((1, _H)),
        _bcast((1, _H)),
        _bcast((_H, _H)),
        _bcast((1, _H)),
        _bcast((1, _H)),
        _bcast((1, _H)),
    ],
    out_specs=[pl.BlockSpec((_BR, _H), lambda i: (i, 0)),
               pl.BlockSpec((1, 1, _H), lambda i: (i, 0, 0))],
    out_shape=[jax.ShapeDtypeStruct((_N, _H), jnp.float32),
               jax.ShapeDtypeStruct((_NBLK, 1, _H), jnp.float32)],
)


def _head_body(h_ref, desc_ref, wh_ref, wd_ref, b1_ref, g_ref, be_ref,
               w2_ref, b2_ref, out_ref):
    hg = jnp.sum(h_ref[...], axis=0, keepdims=True)
    y = (jnp.dot(hg, wh_ref[...], preferred_element_type=jnp.float32)
         + jnp.dot(desc_ref[...], wd_ref[...], preferred_element_type=jnp.float32)
         + b1_ref[...])
    y = jnp.maximum(_ln(y, g_ref[...], be_ref[...]), 0.0)
    out_ref[...] = jnp.dot(y, w2_ref[...], preferred_element_type=jnp.float32) + b2_ref[...]


_head = pl.pallas_call(
    _head_body,
    out_shape=jax.ShapeDtypeStruct((1, _NCLS), jnp.float32),
)


def kernel(x, edge_index, desc, params):
    pk = jnp.bitwise_or(
        jnp.left_shift(edge_index[0], _SHIFT),
        edge_index[1]).reshape(_NW, _EPW)
    h = x
    psum = None
    for l in range(_LAYERS):
        agg = _agg(h, pk)
        eps1 = (1.0 + params['eps_%d' % l]).reshape(1, 1)
        h, psum = _mlp(h, agg, agg, eps1,
                 params['W1_%d' % l], params['b1_%d' % l].reshape(1, _H),
                 params['g1_%d' % l].reshape(1, _H),
                 params['be1_%d' % l].reshape(1, _H),
                 params['W2_%d' % l], params['b2_%d' % l].reshape(1, _H),
                 params['gn_%d' % l].reshape(1, _H),
                 params['bn_%d' % l].reshape(1, _H))
    out = _head(psum.reshape(_NBLK, _H), desc,
                params['fc1_W'][:_H], params['fc1_W'][_H:],
                params['fc1_b'].reshape(1, _H),
                params['n1_g'].reshape(1, _H), params['n1_b'].reshape(1, _H),
                params['fc2_W'], params['fc2_b'].reshape(1, _NCLS))
    return out
